# baseline (speedup 1.0000x reference)
import functools

import jax
import jax.numpy as jnp
from jax import lax
from jax.experimental import pallas as pl
from jax.experimental.pallas import tpu as pltpu
from jax.experimental.pallas import tpu_sc as plsc

NUM_HEADS = 16
H = 32
W = 32
D = 2 * H - 1
TAB = D * D
TAB_PAD = 3976


def _rpe_body(tab_hbm, out_hbm, tab_v, shifted_v, sem):
    c = lax.axis_index("c")
    s = lax.axis_index("s")
    wid = s * 2 + c
    h = wid // 2
    half = wid % 2

    pltpu.sync_copy(tab_hbm.at[h], tab_v)

    def build_x1(x1, carry):
        def build_r(r, carry2):
            b = (62 - r) * D + x1
            v0 = tab_v[pl.ds(b, 16)]
            v1 = tab_v[pl.ds(b + 16, 16)]
            shifted_v[x1, pl.ds(r * 32, 16)] = lax.rev(v1, (0,))
            shifted_v[x1, pl.ds(r * 32 + 16, 16)] = lax.rev(v0, (0,))
            return carry2
        return lax.fori_loop(0, D, build_r, carry)

    lax.fori_loop(0, 32, build_x1, 0)

    copies = []
    for i in range(16):
        y1 = half * 16 + i
        cp = pltpu.make_async_copy(
            shifted_v.at[:, pl.ds((31 - y1) * 32, 1024)],
            out_hbm.at[h, pl.ds(y1 * 32, 32), :],
            sem,
        )
        cp.start()
        copies.append(cp)
    for cp in copies:
        cp.wait()


_rpe_kernel = functools.partial(
    pl.kernel,
    mesh=plsc.VectorSubcoreMesh(core_axis_name="c", subcore_axis_name="s"),
    out_type=jax.ShapeDtypeStruct((NUM_HEADS, H * W, H * W), jnp.float32),
    scratch_types=[
        pltpu.VMEM((TAB_PAD,), jnp.float32),
        pltpu.VMEM((W, D * W), jnp.float32),
        pltpu.SemaphoreType.DMA,
    ],
    compiler_params=pltpu.CompilerParams(use_tc_tiling_on_sc=False),
)(_rpe_body)


def kernel(pos_enc):
    tab = jnp.pad(pos_enc, ((0, 0), (0, TAB_PAD - TAB)))
    return _rpe_kernel(tab)


# speedup vs baseline: 1.1450x; 1.1450x over previous
import functools

import jax
import jax.numpy as jnp
from jax import lax
from jax.experimental import pallas as pl
from jax.experimental.pallas import tpu as pltpu
from jax.experimental.pallas import tpu_sc as plsc

NUM_HEADS = 16
H = 32
W = 32
D = 2 * H - 1
TAB = D * D
ROW_PAD = 4096


def _rpe_body(tab_hbm, out_hbm, tab_v, stg_v, sems):
    c = lax.axis_index("c")
    s = lax.axis_index("s")
    wid = s * 2 + c
    h = wid // 2
    half = wid % 2

    src_off = pl.multiple_of(h * ROW_PAD, ROW_PAD)
    pltpu.sync_copy(tab_hbm.at[pl.ds(src_off, ROW_PAD)], tab_v)

    def build_chunk(y1, buf):
        def build_x1(x1, carry):
            def build_y2(y2, carry2):
                b = (y1 - y2 + 31) * D + x1
                v0 = tab_v[pl.ds(b, 16)]
                v1 = tab_v[pl.ds(b + 16, 16)]
                stg_v[buf, x1, pl.ds(y2 * 32, 16)] = lax.rev(v1, (0,))
                stg_v[buf, x1, pl.ds(y2 * 32 + 16, 16)] = lax.rev(v0, (0,))
                return carry2
            return lax.fori_loop(0, H, build_y2, carry)
        lax.fori_loop(0, W, build_x1, 0)

    def chunk_copy(y1, buf):
        q0 = pl.multiple_of(y1 * W, W)
        return pltpu.make_async_copy(
            stg_v.at[buf],
            out_hbm.at[h, pl.ds(q0, W), :],
            sems.at[buf],
        )

    y1s = [half * 16 + i for i in range(16)]
    build_chunk(y1s[0], 0)
    chunk_copy(y1s[0], 0).start()
    for i in range(1, 16):
        buf = i % 2
        build_chunk(y1s[i], buf)
        chunk_copy(y1s[i - 1], 1 - buf).wait()
        chunk_copy(y1s[i], buf).start()
    chunk_copy(y1s[15], 1).wait()


_rpe_kernel = functools.partial(
    pl.kernel,
    mesh=plsc.VectorSubcoreMesh(core_axis_name="c", subcore_axis_name="s"),
    out_type=jax.ShapeDtypeStruct((NUM_HEADS, H * W, H * W), jnp.float32),
    scratch_types=[
        pltpu.VMEM((ROW_PAD,), jnp.float32),
        pltpu.VMEM((2, W, H * W), jnp.float32),
        pltpu.SemaphoreType.DMA((2,)),
    ],
    compiler_params=pltpu.CompilerParams(use_tc_tiling_on_sc=True),
)(_rpe_body)


def kernel(pos_enc):
    tab = jnp.pad(pos_enc, ((0, 0), (0, ROW_PAD - TAB))).reshape(-1)
    return _rpe_kernel(tab)


# speedup vs baseline: 2.1460x; 1.8743x over previous
import functools

import jax
import jax.numpy as jnp
from jax import lax
from jax.experimental import pallas as pl
from jax.experimental.pallas import tpu as pltpu
from jax.experimental.pallas import tpu_sc as plsc

NUM_HEADS = 16
H = 32
W = 32
D = 2 * H - 1
TAB = D * D
ROW_PAD = 4096
BAND = 44
BAND_COLS = 1536


def _rpe_body(tab_hbm, out_hbm, tab_v, sh_v, sems):
    c = lax.axis_index("c")
    s = lax.axis_index("s")
    wid = s * 2 + c
    h = wid // 2
    base = (wid % 2) * 16

    src_off = pl.multiple_of(h * ROW_PAD, ROW_PAD)
    pltpu.sync_copy(tab_hbm.at[pl.ds(src_off, ROW_PAD)], tab_v)

    def build_band(y1_max, buf):
        def build_x1(x1, carry):
            for m in range(BAND):
                b = (y1_max + 31 - m) * D + x1
                v0 = tab_v[pl.ds(b, 16)]
                v1 = tab_v[pl.ds(b + 16, 16)]
                sh_v[buf, x1, pl.ds(m * 32, 16)] = lax.rev(v1, (0,))
                sh_v[buf, x1, pl.ds(m * 32 + 16, 16)] = lax.rev(v0, (0,))
            return carry
        lax.fori_loop(0, W, build_x1, 0)

    def group_copies(p, buf):
        cps = []
        for k in range(4):
            q0 = pl.multiple_of((base + p + 4 * k) * W, W)
            cps.append(pltpu.make_async_copy(
                sh_v.at[buf, :, pl.ds((3 - k) * 128, H * W)],
                out_hbm.at[h, pl.ds(q0, W), :],
                sems.at[buf],
            ))
        return cps

    inflight = [None, None]
    for p in range(4):
        buf = p % 2
        if inflight[buf] is not None:
            for cp in inflight[buf]:
                cp.wait()
        build_band(base + p + 12, buf)
        cps = group_copies(p, buf)
        for cp in cps:
            cp.start()
        inflight[buf] = cps
    for cps in inflight:
        for cp in cps:
            cp.wait()


_rpe_kernel = functools.partial(
    pl.kernel,
    mesh=plsc.VectorSubcoreMesh(core_axis_name="c", subcore_axis_name="s"),
    out_type=jax.ShapeDtypeStruct((NUM_HEADS, H * W, H * W), jnp.float32),
    scratch_types=[
        pltpu.VMEM((ROW_PAD,), jnp.float32),
        pltpu.VMEM((2, W, BAND_COLS), jnp.float32),
        pltpu.SemaphoreType.DMA((2,)),
    ],
    compiler_params=pltpu.CompilerParams(use_tc_tiling_on_sc=True),
)(_rpe_body)


def kernel(pos_enc):
    tab = jnp.pad(pos_enc, ((0, 0), (0, ROW_PAD - TAB))).reshape(-1)
    return _rpe_kernel(tab)
